# baseline (speedup 1.0000x reference)
import functools

import jax
import jax.numpy as jnp
from jax.experimental import pallas as pl

BN = 512


def _matmul_kernel(x_ref, emb_ref, out_ref):
    out_ref[...] = jax.lax.dot_general(
        x_ref[...], emb_ref[...],
        dimension_numbers=(((1,), (1,)), ((), ())),
        preferred_element_type=jnp.float32,
    )


@jax.jit
def kernel(x, emb_table):
    m, k = x.shape
    vocab, _ = emb_table.shape
    grid = (pl.cdiv(vocab, BN),)
    return pl.pallas_call(
        _matmul_kernel,
        grid=grid,
        in_specs=[
            pl.BlockSpec((m, k), lambda j: (0, 0)),
            pl.BlockSpec((BN, k), lambda j: (j, 0)),
        ],
        out_specs=pl.BlockSpec((m, BN), lambda j: (0, j)),
        out_shape=jax.ShapeDtypeStruct((m, vocab), jnp.float32),
    )(x, emb_table)


# speedup vs baseline: 1.0019x; 1.0019x over previous
import functools

import jax
import jax.numpy as jnp
from jax.experimental import pallas as pl

BN = 1024


def _matmul_kernel(x_ref, emb_ref, out_ref):
    out_ref[...] = jax.lax.dot_general(
        x_ref[...], emb_ref[...],
        dimension_numbers=(((1,), (1,)), ((), ())),
        preferred_element_type=jnp.float32,
    )


@jax.jit
def kernel(x, emb_table):
    m, k = x.shape
    vocab, _ = emb_table.shape
    grid = (pl.cdiv(vocab, BN),)
    return pl.pallas_call(
        _matmul_kernel,
        grid=grid,
        in_specs=[
            pl.BlockSpec((m, k), lambda j: (0, 0)),
            pl.BlockSpec((BN, k), lambda j: (j, 0)),
        ],
        out_specs=pl.BlockSpec((m, BN), lambda j: (0, j)),
        out_shape=jax.ShapeDtypeStruct((m, vocab), jnp.float32),
    )(x, emb_table)


# speedup vs baseline: 1.0021x; 1.0001x over previous
import functools

import jax
import jax.numpy as jnp
from jax.experimental import pallas as pl
from jax.experimental.pallas import tpu as pltpu

BN = 1024


def _matmul_kernel(x_ref, emb_ref, out_ref):
    out_ref[...] = jax.lax.dot_general(
        x_ref[...], emb_ref[...],
        dimension_numbers=(((1,), (1,)), ((), ())),
        preferred_element_type=jnp.float32,
    )


@jax.jit
def kernel(x, emb_table):
    m, k = x.shape
    vocab, _ = emb_table.shape
    grid = (pl.cdiv(vocab, BN),)
    return pl.pallas_call(
        _matmul_kernel,
        grid=grid,
        in_specs=[
            pl.BlockSpec((m, k), lambda j: (0, 0)),
            pl.BlockSpec((BN, k), lambda j: (j, 0)),
        ],
        out_specs=pl.BlockSpec((m, BN), lambda j: (0, j)),
        out_shape=jax.ShapeDtypeStruct((m, vocab), jnp.float32),
        compiler_params=pltpu.CompilerParams(
            dimension_semantics=("parallel",),
        ),
    )(x, emb_table)


# speedup vs baseline: 3.8410x; 3.8330x over previous
import jax
import jax.numpy as jnp
from jax.experimental import pallas as pl
from jax.experimental.pallas import tpu as pltpu

BV = 1024


def _matmul_kernel(emb_ref, x_ref, out_ref):
    out_ref[...] = jax.lax.dot_general(
        emb_ref[...], x_ref[...],
        dimension_numbers=(((1,), (1,)), ((), ())),
        preferred_element_type=jnp.float32,
    )


@jax.jit
def kernel(x, emb_table):
    m, k = x.shape
    vocab, _ = emb_table.shape
    grid = (pl.cdiv(vocab, BV),)
    out_t = pl.pallas_call(
        _matmul_kernel,
        grid=grid,
        in_specs=[
            pl.BlockSpec((BV, k), lambda j: (j, 0)),
            pl.BlockSpec((m, k), lambda j: (0, 0)),
        ],
        out_specs=pl.BlockSpec((BV, m), lambda j: (j, 0)),
        out_shape=jax.ShapeDtypeStruct((vocab, m), jnp.float32),
        compiler_params=pltpu.CompilerParams(
            dimension_semantics=("parallel",),
        ),
    )(emb_table, x)
    return out_t.T
